# baseline (speedup 1.0000x reference)
import functools

import jax
import jax.numpy as jnp
from jax import lax
from jax.experimental import pallas as pl
from jax.experimental.pallas import tpu as pltpu
from jax.experimental.pallas import tpu_sc as plsc

NC = 2
NS = 16
NW = NC * NS
CH = 128


def _sc_mesh():
    return plsc.VectorSubcoreMesh(
        core_axis_name="c", subcore_axis_name="s", num_cores=NC, num_subcores=NS
    )


GDEG = 8


def _make_sc_deg_perm(NP, D, CPD, PC, RPT):
    def body(dstd, ones_h, zeros_h, permp, feat, deg_out, fneg_out,
             deg_sh, didx, ones_v, pidx, frows, dsem, gsem):
        c = lax.axis_index("c")
        s = lax.axis_index("s")
        w = c * NS + s
        row0 = s * RPT
        pltpu.sync_copy(ones_h, ones_v)
        pltpu.sync_copy(zeros_h, deg_sh.at[pl.ds(row0, RPT)])
        pltpu.sync_copy(permp.at[w], pidx.at[pl.ds(0, PC)])
        plsc.subcore_barrier()

        def pb(g, carry):
            pltpu.async_copy(feat.at[pidx.at[g]], frows, gsem).wait()
            pltpu.sync_copy(frows,
                            fneg_out.at[pl.ds(w * PC * CH + g * CH, CH)])
            return carry

        lax.fori_loop(0, PC, pb, 0)

        def gb(g, carry):
            pltpu.sync_copy(dstd.at[w, pl.ds(g * GDEG, GDEG)], didx)

            def jb(j, carry2):
                pltpu.async_copy(ones_v, deg_sh.at[didx.at[j]], dsem, add=True)
                return carry2

            lax.fori_loop(0, GDEG, jb, carry)

            def db(j, carry2):
                pltpu.make_async_copy(ones_v, deg_sh.at[pl.ds(0, CH)],
                                      dsem).wait()
                return carry2

            return lax.fori_loop(0, GDEG, db, carry)

        lax.fori_loop(0, CPD // GDEG, gb, 0)
        plsc.subcore_barrier()
        pltpu.sync_copy(deg_sh.at[pl.ds(row0, RPT)],
                        deg_out.at[pl.ds(c * NP + row0, RPT)])

    return pl.kernel(
        body,
        out_type=[
            jax.ShapeDtypeStruct((NC * NP, D), jnp.float32),
            jax.ShapeDtypeStruct((NW * PC * CH, D), jnp.float32),
        ],
        mesh=_sc_mesh(),
        scratch_types=[
            pltpu.VMEM_SHARED((NP, D), jnp.float32),
            pltpu.VMEM((GDEG, CH), jnp.int32),
            pltpu.VMEM((CH, D), jnp.float32),
            pltpu.VMEM((8, CH), jnp.int32),
            pltpu.VMEM((CH, D), jnp.float32),
            pltpu.SemaphoreType.DMA,
            pltpu.SemaphoreType.DMA,
        ],
    )


GRP = 32


def _make_sc_prop(NP, D, CPT, RPT):
    def body(x2, srcw, dst16, zerosD, agg_out,
             agg_sh, sidx, didx, rows0, rows1, gsem, ssem):
        c = lax.axis_index("c")
        s = lax.axis_index("s")
        w = c * NS + s
        row0 = s * RPT
        pltpu.sync_copy(zerosD, agg_sh.at[pl.ds(row0, RPT)])
        plsc.subcore_barrier()

        def drain_g(buf):
            pltpu.make_async_copy(zerosD.at[pl.ds(0, CH)], buf, gsem).wait()

        def drain_s():
            pltpu.make_async_copy(rows0, agg_sh.at[pl.ds(0, CH)], ssem).wait()

        def gbody(g, carry):
            pltpu.sync_copy(srcw.at[w, pl.ds(g * GRP, GRP)], sidx)
            pltpu.sync_copy(dst16.at[s, pl.ds(g * GRP, GRP)], didx)
            pltpu.async_copy(x2.at[sidx.at[0]], rows0, gsem)

            def pair(t, carry2):
                j = 2 * t
                drain_g(rows0)

                @pl.when(t > 0)
                def _():
                    drain_s()

                pltpu.async_copy(x2.at[sidx.at[j + 1]], rows1, gsem)
                pltpu.async_copy(rows0, agg_sh.at[didx.at[j]], ssem, add=True)
                drain_g(rows1)
                drain_s()

                @pl.when(j + 2 < GRP)
                def _():
                    pltpu.async_copy(x2.at[sidx.at[j + 2]], rows0, gsem)

                pltpu.async_copy(rows1, agg_sh.at[didx.at[j + 1]], ssem,
                                 add=True)
                return carry2

            lax.fori_loop(0, GRP // 2, pair, carry)
            drain_s()
            return carry

        lax.fori_loop(0, CPT // GRP, gbody, 0)
        plsc.subcore_barrier()
        pltpu.sync_copy(agg_sh.at[pl.ds(row0, RPT)],
                        agg_out.at[pl.ds(c * NP + row0, RPT)])

    return pl.kernel(
        body,
        out_type=jax.ShapeDtypeStruct((NC * NP, D), jnp.float32),
        mesh=_sc_mesh(),
        scratch_types=[
            pltpu.VMEM_SHARED((NP, D), jnp.float32),
            pltpu.VMEM((GRP, CH), jnp.int32),
            pltpu.VMEM((GRP, CH), jnp.int32),
            pltpu.VMEM((CH, D), jnp.float32),
            pltpu.VMEM((CH, D), jnp.float32),
            pltpu.SemaphoreType.DMA,
            pltpu.SemaphoreType.DMA,
        ],
    )


def _norm_from(d_ref):
    deg = d_ref[...]
    return jnp.where(deg > 0.0, lax.rsqrt(jnp.maximum(deg, 1.0)), 0.0)


def _tc_prescale(f2, deg2d, NP, D, RPT):
    def body(f_ref, d_ref, o_ref):
        o_ref[...] = f_ref[...] * _norm_from(d_ref)

    nb = (2 * NP) // RPT
    return pl.pallas_call(
        body,
        grid=(nb,),
        in_specs=[
            pl.BlockSpec((RPT, D), lambda i: (i, 0)),
            pl.BlockSpec((RPT, 1), lambda i: (i % (nb // 2), 0)),
        ],
        out_specs=pl.BlockSpec((RPT, D), lambda i: (i, 0)),
        out_shape=jax.ShapeDtypeStruct((2 * NP, D), jnp.float32),
    )(f2, deg2d)


def _tc_layer(agg, deg2d, W, b, NP, D, RPT):
    def body(a_ref, d_ref, w_ref, b_ref, o_ref):
        norm = _norm_from(d_ref)
        h = jnp.dot(a_ref[...] * norm, w_ref[...],
                    preferred_element_type=jnp.float32) + b_ref[...]
        o_ref[...] = jnp.maximum(h, 0.0) * norm

    nb = (2 * NP) // RPT
    return pl.pallas_call(
        body,
        grid=(nb,),
        in_specs=[
            pl.BlockSpec((RPT, D), lambda i: (i, 0)),
            pl.BlockSpec((RPT, 1), lambda i: (i % (nb // 2), 0)),
            pl.BlockSpec((D, D), lambda i: (0, 0)),
            pl.BlockSpec((1, D), lambda i: (0, 0)),
        ],
        out_specs=pl.BlockSpec((RPT, D), lambda i: (i, 0)),
        out_shape=jax.ShapeDtypeStruct((2 * NP, D), jnp.float32),
    )(agg, deg2d, W, b.reshape(1, D))


def _tc_loss(agg2, deg2d, W1, b1, Wd, N, NP, D, RPT):
    nh = NP // RPT

    def body(a_ref, d_ref, w1_ref, b1_ref, wd_ref, o_ref, spos_ref):
        i = pl.program_id(0)
        norm = _norm_from(d_ref)
        rowid = ((i % nh) * RPT
                 + lax.broadcasted_iota(jnp.int32, (RPT, 1), 0))
        real = rowid < N

        @pl.when(i == 0)
        def _():
            spos_ref[...] = jnp.zeros_like(spos_ref)

        @pl.when(i < nh)
        def _():
            nm = jnp.where(real, norm, 0.0)
            spos_ref[...] += jnp.sum(a_ref[...] * nm, axis=0, keepdims=True)

        @pl.when(i >= nh)
        def _():
            summary = jax.nn.sigmoid(
                jnp.dot(spos_ref[...] / N, w1_ref[...],
                        preferred_element_type=jnp.float32) + b1_ref[...])
            ws = lax.dot_general(summary, wd_ref[...],
                                 (((1,), (1,)), ((), ())),
                                 preferred_element_type=jnp.float32)
            h = jnp.dot(a_ref[...] * norm, w1_ref[...],
                        preferred_element_type=jnp.float32) + b1_ref[...]
            logits = lax.dot_general(h, ws, (((1,), (1,)), ((), ())),
                                     preferred_element_type=jnp.float32)
            sign = jnp.where(i < 2 * nh, -1.0, 1.0)
            val = jnp.where(real, jax.nn.softplus(sign * logits), 0.0)
            part = (jnp.sum(val) / N).reshape(1, 1)

            @pl.when(i == nh)
            def _():
                o_ref[...] = jnp.zeros_like(o_ref)

            o_ref[...] += part

    def agg_idx(i):
        return (jnp.where(i < nh, i, i - nh), 0)

    def deg_idx(i):
        return (i % nh, 0)

    return pl.pallas_call(
        body,
        grid=(3 * nh,),
        in_specs=[
            pl.BlockSpec((RPT, D), agg_idx),
            pl.BlockSpec((RPT, 1), deg_idx),
            pl.BlockSpec((D, D), lambda i: (0, 0)),
            pl.BlockSpec((1, D), lambda i: (0, 0)),
            pl.BlockSpec((D, D), lambda i: (0, 0)),
        ],
        out_specs=pl.BlockSpec((1, 1), lambda i: (0, 0)),
        out_shape=jax.ShapeDtypeStruct((1, 1), jnp.float32),
        scratch_shapes=[pltpu.VMEM((1, D), jnp.float32)],
    )(agg2, deg2d, W1, b1.reshape(1, D), Wd)


def kernel(features, edge_index, W0, b0, W1, b1, Wd):
    N, D = features.shape
    E = edge_index.shape[1]
    NP = (N // 256 + 1) * 256
    RPT = NP // NS
    CPT = -(-E // (NS * CH * GRP)) * GRP
    CPD = -(-E // (NW * CH * GDEG)) * GDEG
    PC = -(-(NP // NW) // CH)

    src = edge_index[0].astype(jnp.int32)
    dst = edge_index[1].astype(jnp.int32)
    perm = jax.random.permutation(jax.random.key(42), N).astype(jnp.int32)

    src16 = jnp.pad(src, (0, NS * CPT * CH - E)).reshape(NS, CPT, CH)
    srcw = jnp.concatenate([src16, src16 + NP]).reshape(NW, CPT, CH)
    dst16 = jnp.pad(dst, (0, NS * CPT * CH - E),
                    constant_values=N).reshape(NS, CPT, CH)
    dstd = jnp.pad(dst, (0, NW * CPD * CH - E),
                   constant_values=N).reshape(NW, CPD, CH)
    permp = jnp.pad(perm, (0, NW * PC * CH - N)).reshape(NW, PC, CH)
    zerosD = jnp.zeros((RPT, D), jnp.float32)
    onesD = jnp.ones((CH, D), jnp.float32)

    deg2, fneg_raw = _make_sc_deg_perm(NP, D, CPD, PC, RPT)(
        dstd, onesD, zerosD, permp, features)
    deg2d = (deg2[:NP, 0] + deg2[NP:, 0]).reshape(NP, 1)

    fpad = jnp.pad(features, ((0, NP - N), (0, 0)))
    f2 = jnp.concatenate([fpad, fneg_raw[:NP]], axis=0)

    prop = _make_sc_prop(NP, D, CPT, RPT)
    x0 = _tc_prescale(f2, deg2d, NP, D, RPT)
    agg1 = prop(x0, srcw, dst16, zerosD)
    x1 = _tc_layer(agg1, deg2d, W0, b0, NP, D, RPT)
    agg2 = prop(x1, srcw, dst16, zerosD)
    loss = _tc_loss(agg2, deg2d, W1, b1, Wd, N, NP, D, RPT)
    return loss[0, 0]


# speedup vs baseline: 1.1025x; 1.1025x over previous
import functools

import jax
import jax.numpy as jnp
from jax import lax
from jax.experimental import pallas as pl
from jax.experimental.pallas import tpu as pltpu
from jax.experimental.pallas import tpu_sc as plsc

NC = 2
NS = 16
NW = NC * NS
CH = 128


def _sc_mesh():
    return plsc.VectorSubcoreMesh(
        core_axis_name="c", subcore_axis_name="s", num_cores=NC, num_subcores=NS
    )


GDEG = 8


def _make_sc_deg_perm(NP, D, CPD, PC, RPT):
    def body(dstd, ones_h, zeros_h, deg_out, deg_sh, didx, ones_v, sem):
        c = lax.axis_index("c")
        s = lax.axis_index("s")
        w = c * NS + s
        row0 = s * RPT
        pltpu.sync_copy(ones_h, ones_v)
        pltpu.sync_copy(zeros_h, deg_sh.at[pl.ds(row0, RPT)])
        plsc.subcore_barrier()

        def gb(g, carry):
            pltpu.sync_copy(dstd.at[w, pl.ds(g * GDEG, GDEG)], didx)

            def jb(j, carry2):
                pltpu.async_copy(ones_v, deg_sh.at[didx.at[j]], sem, add=True)
                return carry2

            lax.fori_loop(0, GDEG, jb, carry)

            def db(j, carry2):
                pltpu.make_async_copy(ones_v, deg_sh.at[pl.ds(0, CH)],
                                      sem).wait()
                return carry2

            return lax.fori_loop(0, GDEG, db, carry)

        lax.fori_loop(0, CPD // GDEG, gb, 0)
        plsc.subcore_barrier()
        pltpu.sync_copy(deg_sh.at[pl.ds(row0, RPT)],
                        deg_out.at[pl.ds(c * NP + row0, RPT)])

    return pl.kernel(
        body,
        out_type=jax.ShapeDtypeStruct((NC * NP, D), jnp.float32),
        mesh=_sc_mesh(),
        scratch_types=[
            pltpu.VMEM_SHARED((NP, D), jnp.float32),
            pltpu.VMEM((GDEG, CH), jnp.int32),
            pltpu.VMEM((CH, D), jnp.float32),
            pltpu.SemaphoreType.DMA,
        ],
    )


def _make_sc_perm(NP, D, PC):
    def body(permp, feat, fneg_out, pidx, fr0, fr1, sem):
        c = lax.axis_index("c")
        s = lax.axis_index("s")
        w = c * NS + s
        pltpu.sync_copy(permp.at[w], pidx.at[pl.ds(0, PC)])
        bufs = [fr0, fr1]
        for j in range(PC):
            fr = bufs[j % 2]
            pltpu.async_copy(feat.at[pidx.at[j]], fr, sem).wait()
            pltpu.sync_copy(fr, fneg_out.at[pl.ds(w * PC * CH + j * CH, CH)])

    return pl.kernel(
        body,
        out_type=jax.ShapeDtypeStruct((NW * PC * CH, D), jnp.float32),
        mesh=_sc_mesh(),
        scratch_types=[
            pltpu.VMEM((8, CH), jnp.int32),
            pltpu.VMEM((CH, D), jnp.float32),
            pltpu.VMEM((CH, D), jnp.float32),
            pltpu.SemaphoreType.DMA,
        ],
    )


GRP = 32


def _make_sc_prop(NP, D, CPT, RPT):
    def body(x2, srcw, dst16, zerosD, agg_out,
             agg_sh, sidx, didx, rows0, rows1, gsem, ssem):
        c = lax.axis_index("c")
        s = lax.axis_index("s")
        w = c * NS + s
        row0 = s * RPT
        pltpu.sync_copy(zerosD, agg_sh.at[pl.ds(row0, RPT)])
        plsc.subcore_barrier()

        def drain_g(buf):
            pltpu.make_async_copy(zerosD.at[pl.ds(0, CH)], buf, gsem).wait()

        def drain_s():
            pltpu.make_async_copy(rows0, agg_sh.at[pl.ds(0, CH)], ssem).wait()

        def gbody(g, carry):
            pltpu.sync_copy(srcw.at[w, pl.ds(g * GRP, GRP)], sidx)
            pltpu.sync_copy(dst16.at[s, pl.ds(g * GRP, GRP)], didx)
            pltpu.async_copy(x2.at[sidx.at[0]], rows0, gsem)

            def pair(t, carry2):
                j = 2 * t
                drain_g(rows0)

                @pl.when(t > 0)
                def _():
                    drain_s()

                pltpu.async_copy(x2.at[sidx.at[j + 1]], rows1, gsem)
                pltpu.async_copy(rows0, agg_sh.at[didx.at[j]], ssem, add=True)
                drain_g(rows1)
                drain_s()

                @pl.when(j + 2 < GRP)
                def _():
                    pltpu.async_copy(x2.at[sidx.at[j + 2]], rows0, gsem)

                pltpu.async_copy(rows1, agg_sh.at[didx.at[j + 1]], ssem,
                                 add=True)
                return carry2

            lax.fori_loop(0, GRP // 2, pair, carry)
            drain_s()
            return carry

        lax.fori_loop(0, CPT // GRP, gbody, 0)
        plsc.subcore_barrier()
        pltpu.sync_copy(agg_sh.at[pl.ds(row0, RPT)],
                        agg_out.at[pl.ds(c * NP + row0, RPT)])

    return pl.kernel(
        body,
        out_type=jax.ShapeDtypeStruct((NC * NP, D), jnp.float32),
        mesh=_sc_mesh(),
        scratch_types=[
            pltpu.VMEM_SHARED((NP, D), jnp.float32),
            pltpu.VMEM((GRP, CH), jnp.int32),
            pltpu.VMEM((GRP, CH), jnp.int32),
            pltpu.VMEM((CH, D), jnp.float32),
            pltpu.VMEM((CH, D), jnp.float32),
            pltpu.SemaphoreType.DMA,
            pltpu.SemaphoreType.DMA,
        ],
    )


def _norm_from(d_ref):
    deg = d_ref[...]
    return jnp.where(deg > 0.0, lax.rsqrt(jnp.maximum(deg, 1.0)), 0.0)


def _tc_prescale(f2, deg2d, NP, D, RPT):
    def body(f_ref, d_ref, o_ref):
        o_ref[...] = f_ref[...] * _norm_from(d_ref)

    nb = (2 * NP) // RPT
    return pl.pallas_call(
        body,
        grid=(nb,),
        in_specs=[
            pl.BlockSpec((RPT, D), lambda i: (i, 0)),
            pl.BlockSpec((RPT, 1), lambda i: (i % (nb // 2), 0)),
        ],
        out_specs=pl.BlockSpec((RPT, D), lambda i: (i, 0)),
        out_shape=jax.ShapeDtypeStruct((2 * NP, D), jnp.float32),
    )(f2, deg2d)


def _tc_layer(agg, deg2d, W, b, NP, D, RPT):
    def body(a_ref, d_ref, w_ref, b_ref, o_ref):
        norm = _norm_from(d_ref)
        h = jnp.dot(a_ref[...] * norm, w_ref[...],
                    preferred_element_type=jnp.float32) + b_ref[...]
        o_ref[...] = jnp.maximum(h, 0.0) * norm

    nb = (2 * NP) // RPT
    return pl.pallas_call(
        body,
        grid=(nb,),
        in_specs=[
            pl.BlockSpec((RPT, D), lambda i: (i, 0)),
            pl.BlockSpec((RPT, 1), lambda i: (i % (nb // 2), 0)),
            pl.BlockSpec((D, D), lambda i: (0, 0)),
            pl.BlockSpec((1, D), lambda i: (0, 0)),
        ],
        out_specs=pl.BlockSpec((RPT, D), lambda i: (i, 0)),
        out_shape=jax.ShapeDtypeStruct((2 * NP, D), jnp.float32),
    )(agg, deg2d, W, b.reshape(1, D))


def _tc_loss(agg2, deg2d, W1, b1, Wd, N, NP, D, RPT):
    nh = NP // RPT

    def body(a_ref, d_ref, w1_ref, b1_ref, wd_ref, o_ref, spos_ref):
        i = pl.program_id(0)
        norm = _norm_from(d_ref)
        rowid = ((i % nh) * RPT
                 + lax.broadcasted_iota(jnp.int32, (RPT, 1), 0))
        real = rowid < N

        @pl.when(i == 0)
        def _():
            spos_ref[...] = jnp.zeros_like(spos_ref)

        @pl.when(i < nh)
        def _():
            nm = jnp.where(real, norm, 0.0)
            spos_ref[...] += jnp.sum(a_ref[...] * nm, axis=0, keepdims=True)

        @pl.when(i >= nh)
        def _():
            summary = jax.nn.sigmoid(
                jnp.dot(spos_ref[...] / N, w1_ref[...],
                        preferred_element_type=jnp.float32) + b1_ref[...])
            ws = lax.dot_general(summary, wd_ref[...],
                                 (((1,), (1,)), ((), ())),
                                 preferred_element_type=jnp.float32)
            h = jnp.dot(a_ref[...] * norm, w1_ref[...],
                        preferred_element_type=jnp.float32) + b1_ref[...]
            logits = lax.dot_general(h, ws, (((1,), (1,)), ((), ())),
                                     preferred_element_type=jnp.float32)
            sign = jnp.where(i < 2 * nh, -1.0, 1.0)
            val = jnp.where(real, jax.nn.softplus(sign * logits), 0.0)
            part = (jnp.sum(val) / N).reshape(1, 1)

            @pl.when(i == nh)
            def _():
                o_ref[...] = jnp.zeros_like(o_ref)

            o_ref[...] += part

    def agg_idx(i):
        return (jnp.where(i < nh, i, i - nh), 0)

    def deg_idx(i):
        return (i % nh, 0)

    return pl.pallas_call(
        body,
        grid=(3 * nh,),
        in_specs=[
            pl.BlockSpec((RPT, D), agg_idx),
            pl.BlockSpec((RPT, 1), deg_idx),
            pl.BlockSpec((D, D), lambda i: (0, 0)),
            pl.BlockSpec((1, D), lambda i: (0, 0)),
            pl.BlockSpec((D, D), lambda i: (0, 0)),
        ],
        out_specs=pl.BlockSpec((1, 1), lambda i: (0, 0)),
        out_shape=jax.ShapeDtypeStruct((1, 1), jnp.float32),
        scratch_shapes=[pltpu.VMEM((1, D), jnp.float32)],
    )(agg2, deg2d, W1, b1.reshape(1, D), Wd)


def kernel(features, edge_index, W0, b0, W1, b1, Wd):
    N, D = features.shape
    E = edge_index.shape[1]
    NP = (N // 256 + 1) * 256
    RPT = NP // NS
    CPT = -(-E // (NS * CH * GRP)) * GRP
    CPD = -(-E // (NW * CH * GDEG)) * GDEG
    PC = -(-(NP // NW) // CH)

    src = edge_index[0].astype(jnp.int32)
    dst = edge_index[1].astype(jnp.int32)
    perm = jax.random.permutation(jax.random.key(42), N).astype(jnp.int32)

    src16 = jnp.pad(src, (0, NS * CPT * CH - E)).reshape(NS, CPT, CH)
    srcw = jnp.concatenate([src16, src16 + NP]).reshape(NW, CPT, CH)
    dst16 = jnp.pad(dst, (0, NS * CPT * CH - E),
                    constant_values=N).reshape(NS, CPT, CH)
    dstd = jnp.pad(dst, (0, NW * CPD * CH - E),
                   constant_values=N).reshape(NW, CPD, CH)
    permp = jnp.pad(perm, (0, NW * PC * CH - N)).reshape(NW, PC, CH)
    zerosD = jnp.zeros((RPT, D), jnp.float32)
    onesD = jnp.ones((CH, D), jnp.float32)

    deg2 = _make_sc_deg_perm(NP, D, CPD, PC, RPT)(dstd, onesD, zerosD)
    fneg_raw = _make_sc_perm(NP, D, PC)(permp, features)
    deg2d = (deg2[:NP, 0] + deg2[NP:, 0]).reshape(NP, 1)

    fpad = jnp.pad(features, ((0, NP - N), (0, 0)))
    f2 = jnp.concatenate([fpad, fneg_raw[:NP]], axis=0)

    prop = _make_sc_prop(NP, D, CPT, RPT)
    x0 = _tc_prescale(f2, deg2d, NP, D, RPT)
    agg1 = prop(x0, srcw, dst16, zerosD)
    x1 = _tc_layer(agg1, deg2d, W0, b0, NP, D, RPT)
    agg2 = prop(x1, srcw, dst16, zerosD)
    loss = _tc_loss(agg2, deg2d, W1, b1, Wd, N, NP, D, RPT)
    return loss[0, 0]


# speedup vs baseline: 1.1243x; 1.0198x over previous
import functools

import jax
import jax.numpy as jnp
from jax import lax
from jax.experimental import pallas as pl
from jax.experimental.pallas import tpu as pltpu
from jax.experimental.pallas import tpu_sc as plsc

NC = 2
NS = 16
NW = NC * NS
CH = 128


def _sc_mesh():
    return plsc.VectorSubcoreMesh(
        core_axis_name="c", subcore_axis_name="s", num_cores=NC, num_subcores=NS
    )


GDEG = 8


def _make_sc_deg_perm(NP, D, CPD, PC, RPT):
    def body(dstd, ones_h, zeros_h, deg_out, deg_sh, didx, ones_v, sem):
        c = lax.axis_index("c")
        s = lax.axis_index("s")
        w = c * NS + s
        row0 = s * RPT
        pltpu.sync_copy(ones_h, ones_v)
        pltpu.sync_copy(zeros_h, deg_sh.at[pl.ds(row0, RPT)])
        plsc.subcore_barrier()

        def gb(g, carry):
            pltpu.sync_copy(dstd.at[w, pl.ds(g * GDEG, GDEG)], didx)

            def jb(j, carry2):
                pltpu.async_copy(ones_v, deg_sh.at[didx.at[j]], sem, add=True)
                return carry2

            lax.fori_loop(0, GDEG, jb, carry)

            def db(j, carry2):
                pltpu.make_async_copy(ones_v, deg_sh.at[pl.ds(0, CH)],
                                      sem).wait()
                return carry2

            return lax.fori_loop(0, GDEG, db, carry)

        lax.fori_loop(0, CPD // GDEG, gb, 0)
        plsc.subcore_barrier()
        pltpu.sync_copy(deg_sh.at[pl.ds(row0, RPT)],
                        deg_out.at[pl.ds(c * NP + row0, RPT)])

    return pl.kernel(
        body,
        out_type=jax.ShapeDtypeStruct((NC * NP, D), jnp.float32),
        mesh=_sc_mesh(),
        scratch_types=[
            pltpu.VMEM_SHARED((NP, D), jnp.float32),
            pltpu.VMEM((GDEG, CH), jnp.int32),
            pltpu.VMEM((CH, D), jnp.float32),
            pltpu.SemaphoreType.DMA,
        ],
    )


def _make_sc_perm(NP, D, PC):
    def body(permp, feat, fneg_out, pidx, fr0, fr1, sem):
        c = lax.axis_index("c")
        s = lax.axis_index("s")
        w = c * NS + s
        pltpu.sync_copy(permp.at[w], pidx.at[pl.ds(0, PC)])
        bufs = [fr0, fr1]
        for j in range(PC):
            fr = bufs[j % 2]
            pltpu.async_copy(feat.at[pidx.at[j]], fr, sem).wait()
            pltpu.sync_copy(fr, fneg_out.at[pl.ds(w * PC * CH + j * CH, CH)])

    return pl.kernel(
        body,
        out_type=jax.ShapeDtypeStruct((NW * PC * CH, D), jnp.float32),
        mesh=_sc_mesh(),
        scratch_types=[
            pltpu.VMEM((8, CH), jnp.int32),
            pltpu.VMEM((CH, D), jnp.float32),
            pltpu.VMEM((CH, D), jnp.float32),
            pltpu.SemaphoreType.DMA,
        ],
    )


GRP = 32


def _make_sc_prop(NP, D, CPT, RPT):
    def body(x2, srcw, dst16, zerosD, agg_out,
             agg_sh, sidx, didx, rows0, rows1, gsem, ssem):
        c = lax.axis_index("c")
        s = lax.axis_index("s")
        w = c * NS + s
        row0 = s * RPT
        pltpu.sync_copy(zerosD, agg_sh.at[pl.ds(row0, RPT)])
        plsc.subcore_barrier()

        def drain_g(buf):
            pltpu.make_async_copy(zerosD.at[pl.ds(0, CH)], buf, gsem).wait()

        def drain_s():
            pltpu.make_async_copy(rows0, agg_sh.at[pl.ds(0, CH)], ssem).wait()

        def gbody(g, carry):
            pltpu.sync_copy(srcw.at[w, pl.ds(g * GRP, GRP)], sidx)
            pltpu.sync_copy(dst16.at[s, pl.ds(g * GRP, GRP)], didx)
            pltpu.async_copy(x2.at[sidx.at[0]], rows0, gsem)

            def pair(t, carry2):
                j = 2 * t
                drain_g(rows0)

                @pl.when(t > 0)
                def _():
                    drain_s()

                pltpu.async_copy(x2.at[sidx.at[j + 1]], rows1, gsem)
                pltpu.async_copy(rows0, agg_sh.at[didx.at[j]], ssem, add=True)
                drain_g(rows1)
                drain_s()

                @pl.when(j + 2 < GRP)
                def _():
                    pltpu.async_copy(x2.at[sidx.at[j + 2]], rows0, gsem)

                pltpu.async_copy(rows1, agg_sh.at[didx.at[j + 1]], ssem,
                                 add=True)
                return carry2

            lax.fori_loop(0, GRP // 2, pair, carry)
            drain_s()
            return carry

        lax.fori_loop(0, CPT // GRP, gbody, 0)
        plsc.subcore_barrier()
        pltpu.sync_copy(agg_sh.at[pl.ds(row0, RPT)],
                        agg_out.at[pl.ds(c * NP + row0, RPT)])

    return pl.kernel(
        body,
        out_type=jax.ShapeDtypeStruct((NC * NP, D), jnp.float32),
        mesh=_sc_mesh(),
        scratch_types=[
            pltpu.VMEM_SHARED((NP, D), jnp.float32),
            pltpu.VMEM((GRP, CH), jnp.int32),
            pltpu.VMEM((GRP, CH), jnp.int32),
            pltpu.VMEM((CH, D), jnp.float32),
            pltpu.VMEM((CH, D), jnp.float32),
            pltpu.SemaphoreType.DMA,
            pltpu.SemaphoreType.DMA,
        ],
    )


def _norm_from(d_ref):
    deg = d_ref[...]
    return jnp.where(deg > 0.0, lax.rsqrt(jnp.maximum(deg, 1.0)), 0.0)


def _tc_prescale(f2, deg2d, NP, D, RPT):
    def body(f_ref, d_ref, o_ref):
        o_ref[...] = f_ref[...] * _norm_from(d_ref)

    nb = (2 * NP) // RPT
    return pl.pallas_call(
        body,
        grid=(nb,),
        in_specs=[
            pl.BlockSpec((RPT, D), lambda i: (i, 0)),
            pl.BlockSpec((RPT, 1), lambda i: (i % (nb // 2), 0)),
        ],
        out_specs=pl.BlockSpec((RPT, D), lambda i: (i, 0)),
        out_shape=jax.ShapeDtypeStruct((2 * NP, D), jnp.float32),
    )(f2, deg2d)


def _tc_layer(agg, deg2d, W, b, NP, D, RPT):
    def body(a_ref, d_ref, w_ref, b_ref, o_ref):
        norm = _norm_from(d_ref)
        h = jnp.dot(a_ref[...] * norm, w_ref[...],
                    preferred_element_type=jnp.float32) + b_ref[...]
        o_ref[...] = jnp.maximum(h, 0.0) * norm

    nb = (2 * NP) // RPT
    return pl.pallas_call(
        body,
        grid=(nb,),
        in_specs=[
            pl.BlockSpec((RPT, D), lambda i: (i, 0)),
            pl.BlockSpec((RPT, 1), lambda i: (i % (nb // 2), 0)),
            pl.BlockSpec((D, D), lambda i: (0, 0)),
            pl.BlockSpec((1, D), lambda i: (0, 0)),
        ],
        out_specs=pl.BlockSpec((RPT, D), lambda i: (i, 0)),
        out_shape=jax.ShapeDtypeStruct((2 * NP, D), jnp.float32),
    )(agg, deg2d, W, b.reshape(1, D))


def _tc_loss(agg2, deg2d, W1, b1, Wd, N, NP, D, RPT):
    nh = NP // RPT

    def body(a_ref, d_ref, w1_ref, b1_ref, wd_ref, o_ref, spos_ref):
        i = pl.program_id(0)
        norm = _norm_from(d_ref)
        rowid = ((i % nh) * RPT
                 + lax.broadcasted_iota(jnp.int32, (RPT, 1), 0))
        real = rowid < N

        @pl.when(i == 0)
        def _():
            spos_ref[...] = jnp.zeros_like(spos_ref)

        @pl.when(i < nh)
        def _():
            nm = jnp.where(real, norm, 0.0)
            spos_ref[...] += jnp.sum(a_ref[...] * nm, axis=0, keepdims=True)

        @pl.when(i >= nh)
        def _():
            summary = jax.nn.sigmoid(
                jnp.dot(spos_ref[...] / N, w1_ref[...],
                        preferred_element_type=jnp.float32) + b1_ref[...])
            ws = lax.dot_general(summary, wd_ref[...],
                                 (((1,), (1,)), ((), ())),
                                 preferred_element_type=jnp.float32)
            h = jnp.dot(a_ref[...] * norm, w1_ref[...],
                        preferred_element_type=jnp.float32) + b1_ref[...]
            logits = lax.dot_general(h, ws, (((1,), (1,)), ((), ())),
                                     preferred_element_type=jnp.float32)
            sign = jnp.where(i < 2 * nh, -1.0, 1.0)
            val = jnp.where(real, jax.nn.softplus(sign * logits), 0.0)
            part = (jnp.sum(val) / N).reshape(1, 1)

            @pl.when(i == nh)
            def _():
                o_ref[...] = jnp.zeros_like(o_ref)

            o_ref[...] += part

    def agg_idx(i):
        return (jnp.where(i < nh, i, i - nh), 0)

    def deg_idx(i):
        return (i % nh, 0)

    return pl.pallas_call(
        body,
        grid=(3 * nh,),
        in_specs=[
            pl.BlockSpec((RPT, D), agg_idx),
            pl.BlockSpec((RPT, 1), deg_idx),
            pl.BlockSpec((D, D), lambda i: (0, 0)),
            pl.BlockSpec((1, D), lambda i: (0, 0)),
            pl.BlockSpec((D, D), lambda i: (0, 0)),
        ],
        out_specs=pl.BlockSpec((1, 1), lambda i: (0, 0)),
        out_shape=jax.ShapeDtypeStruct((1, 1), jnp.float32),
        scratch_shapes=[pltpu.VMEM((1, D), jnp.float32)],
    )(agg2, deg2d, W1, b1.reshape(1, D), Wd)


def kernel(features, edge_index, W0, b0, W1, b1, Wd):
    N, D = features.shape
    E = edge_index.shape[1]
    NP = (N // 256 + 1) * 256
    RPT = NP // NS
    CPT = -(-E // (NS * CH * GRP)) * GRP
    CPD = -(-E // (NW * CH * GDEG)) * GDEG
    PC = -(-(NP // NW) // CH)

    src = edge_index[0].astype(jnp.int32)
    dst = edge_index[1].astype(jnp.int32)
    perm = jax.random.permutation(jax.random.key(42), N).astype(jnp.int32)

    src16 = jnp.pad(src, (0, NS * CPT * CH - E)).reshape(NS, CPT, CH)
    srcw = jnp.concatenate([src16, src16 + NP]).reshape(NW, CPT, CH)
    dst16 = jnp.pad(dst, (0, NS * CPT * CH - E),
                    constant_values=N).reshape(NS, CPT, CH)
    dstd = jnp.pad(dst, (0, NW * CPD * CH - E),
                   constant_values=N).reshape(NW, CPD, CH)
    permp = jnp.pad(perm, (0, NW * PC * CH - N)).reshape(NW, PC, CH)
    zerosD = jnp.zeros((RPT, D), jnp.float32)
    onesD = jnp.ones((CH, D), jnp.float32)

    deg2 = _make_sc_deg_perm(NP, D, CPD, PC, RPT)(dstd, onesD, zerosD)
    fneg_raw = _make_sc_perm(NP, D, PC)(permp, features)
    deg2d = (deg2[:NP, 0] + deg2[NP:, 0]).reshape(NP, 1)

    fpad = jnp.pad(features, ((0, NP - N), (0, 0)))
    f2 = jnp.concatenate([fpad, fneg_raw[:NP]], axis=0)

    RB = 2 * RPT
    prop = _make_sc_prop(NP, D, CPT, RPT)
    x0 = _tc_prescale(f2, deg2d, NP, D, RB)
    agg1 = prop(x0, srcw, dst16, zerosD)
    x1 = _tc_layer(agg1, deg2d, W0, b0, NP, D, RB)
    agg2 = prop(x1, srcw, dst16, zerosD)
    loss = _tc_loss(agg2, deg2d, W1, b1, Wd, N, NP, D, RB)
    return loss[0, 0]


# speedup vs baseline: 1.1338x; 1.0085x over previous
import functools

import jax
import jax.numpy as jnp
from jax import lax
from jax.experimental import pallas as pl
from jax.experimental.pallas import tpu as pltpu
from jax.experimental.pallas import tpu_sc as plsc

NC = 2
NS = 16
NW = NC * NS
CH = 128


def _sc_mesh():
    return plsc.VectorSubcoreMesh(
        core_axis_name="c", subcore_axis_name="s", num_cores=NC, num_subcores=NS
    )


GDEG = 8


def _make_sc_deg_perm(NP, D, CPD, PC, RPT):
    def body(dstd, ones_h, zeros_h, deg_out, deg_sh, didx, ones_v, sem):
        c = lax.axis_index("c")
        s = lax.axis_index("s")
        w = c * NS + s
        row0 = s * RPT
        pltpu.sync_copy(ones_h, ones_v)
        pltpu.sync_copy(zeros_h, deg_sh.at[pl.ds(row0, RPT)])
        plsc.subcore_barrier()

        def gb(g, carry):
            pltpu.sync_copy(dstd.at[w, pl.ds(g * GDEG, GDEG)], didx)

            def jb(j, carry2):
                pltpu.async_copy(ones_v, deg_sh.at[didx.at[j]], sem, add=True)
                return carry2

            lax.fori_loop(0, GDEG, jb, carry)

            def db(j, carry2):
                pltpu.make_async_copy(ones_v, deg_sh.at[pl.ds(0, CH)],
                                      sem).wait()
                return carry2

            return lax.fori_loop(0, GDEG, db, carry)

        lax.fori_loop(0, CPD // GDEG, gb, 0)
        plsc.subcore_barrier()
        pltpu.sync_copy(deg_sh.at[pl.ds(row0, RPT)],
                        deg_out.at[pl.ds(c * NP + row0, RPT)])

    return pl.kernel(
        body,
        out_type=jax.ShapeDtypeStruct((NC * NP, D), jnp.float32),
        mesh=_sc_mesh(),
        scratch_types=[
            pltpu.VMEM_SHARED((NP, D), jnp.float32),
            pltpu.VMEM((GDEG, CH), jnp.int32),
            pltpu.VMEM((CH, D), jnp.float32),
            pltpu.SemaphoreType.DMA,
        ],
    )


def _make_sc_perm(NP, D, PC):
    def body(permp, feat, fneg_out, pidx, fr0, fr1, sem):
        c = lax.axis_index("c")
        s = lax.axis_index("s")
        w = c * NS + s
        pltpu.sync_copy(permp.at[w], pidx.at[pl.ds(0, PC)])
        bufs = [fr0, fr1]
        for j in range(PC):
            fr = bufs[j % 2]
            pltpu.async_copy(feat.at[pidx.at[j]], fr, sem).wait()
            pltpu.sync_copy(fr, fneg_out.at[pl.ds(w * PC * CH + j * CH, CH)])

    return pl.kernel(
        body,
        out_type=jax.ShapeDtypeStruct((NW * PC * CH, D), jnp.float32),
        mesh=_sc_mesh(),
        scratch_types=[
            pltpu.VMEM((8, CH), jnp.int32),
            pltpu.VMEM((CH, D), jnp.float32),
            pltpu.VMEM((CH, D), jnp.float32),
            pltpu.SemaphoreType.DMA,
        ],
    )


GRP = 32


def _make_sc_prop(NP, D, CPT, RPT):
    def body(x2, srcw, dst16, zerosD, agg_out,
             agg_sh, sidx, didx, rows0, rows1, gsem, ssem):
        c = lax.axis_index("c")
        s = lax.axis_index("s")
        w = c * NS + s
        row0 = s * RPT
        pltpu.sync_copy(zerosD, agg_sh.at[pl.ds(row0, RPT)])
        plsc.subcore_barrier()

        def drain_g(buf):
            pltpu.make_async_copy(zerosD.at[pl.ds(0, CH)], buf, gsem).wait()

        def drain_s():
            pltpu.make_async_copy(rows0, agg_sh.at[pl.ds(0, CH)], ssem).wait()

        def gbody(g, carry):
            pltpu.sync_copy(srcw.at[w, pl.ds(g * GRP, GRP)], sidx)
            pltpu.sync_copy(dst16.at[s, pl.ds(g * GRP, GRP)], didx)
            pltpu.async_copy(x2.at[sidx.at[0]], rows0, gsem)

            def pair(t, carry2):
                j = 2 * t
                drain_g(rows0)

                @pl.when(t > 0)
                def _():
                    drain_s()

                pltpu.async_copy(x2.at[sidx.at[j + 1]], rows1, gsem)
                pltpu.async_copy(rows0, agg_sh.at[didx.at[j]], ssem, add=True)
                drain_g(rows1)
                drain_s()

                @pl.when(j + 2 < GRP)
                def _():
                    pltpu.async_copy(x2.at[sidx.at[j + 2]], rows0, gsem)

                pltpu.async_copy(rows1, agg_sh.at[didx.at[j + 1]], ssem,
                                 add=True)
                return carry2

            lax.fori_loop(0, GRP // 2, pair, carry)
            drain_s()
            return carry

        lax.fori_loop(0, CPT // GRP, gbody, 0)
        plsc.subcore_barrier()
        pltpu.sync_copy(agg_sh.at[pl.ds(row0, RPT)],
                        agg_out.at[pl.ds(c * NP + row0, RPT)])

    return pl.kernel(
        body,
        out_type=jax.ShapeDtypeStruct((NC * NP, D), jnp.float32),
        mesh=_sc_mesh(),
        scratch_types=[
            pltpu.VMEM_SHARED((NP, D), jnp.float32),
            pltpu.VMEM((GRP, CH), jnp.int32),
            pltpu.VMEM((GRP, CH), jnp.int32),
            pltpu.VMEM((CH, D), jnp.float32),
            pltpu.VMEM((CH, D), jnp.float32),
            pltpu.SemaphoreType.DMA,
            pltpu.SemaphoreType.DMA,
        ],
    )


def _norm_from(d_ref):
    deg = d_ref[...]
    return jnp.where(deg > 0.0, lax.rsqrt(jnp.maximum(deg, 1.0)), 0.0)


def _tc_prescale(f2, deg2d, NP, D, RPT):
    def body(f_ref, d_ref, o_ref):
        o_ref[...] = f_ref[...] * _norm_from(d_ref)

    nb = (2 * NP) // RPT
    return pl.pallas_call(
        body,
        grid=(nb,),
        in_specs=[
            pl.BlockSpec((RPT, D), lambda i: (i, 0)),
            pl.BlockSpec((RPT, 1), lambda i: (i % (nb // 2), 0)),
        ],
        out_specs=pl.BlockSpec((RPT, D), lambda i: (i, 0)),
        out_shape=jax.ShapeDtypeStruct((2 * NP, D), jnp.float32),
    )(f2, deg2d)


def _tc_layer(agg, deg2d, W, b, NP, D, RPT):
    def body(a_ref, d_ref, w_ref, b_ref, o_ref):
        norm = _norm_from(d_ref)
        h = jnp.dot(a_ref[...] * norm, w_ref[...],
                    preferred_element_type=jnp.float32) + b_ref[...]
        o_ref[...] = jnp.maximum(h, 0.0) * norm

    nb = (2 * NP) // RPT
    return pl.pallas_call(
        body,
        grid=(nb,),
        in_specs=[
            pl.BlockSpec((RPT, D), lambda i: (i, 0)),
            pl.BlockSpec((RPT, 1), lambda i: (i % (nb // 2), 0)),
            pl.BlockSpec((D, D), lambda i: (0, 0)),
            pl.BlockSpec((1, D), lambda i: (0, 0)),
        ],
        out_specs=pl.BlockSpec((RPT, D), lambda i: (i, 0)),
        out_shape=jax.ShapeDtypeStruct((2 * NP, D), jnp.float32),
    )(agg, deg2d, W, b.reshape(1, D))


def _tc_loss(agg2, deg2d, W1, b1, Wd, N, NP, D, RPT):
    nh = NP // RPT

    def body(a_ref, d_ref, w1_ref, b1_ref, wd_ref, o_ref, spos_ref):
        i = pl.program_id(0)
        norm = _norm_from(d_ref)
        rowid = ((i % nh) * RPT
                 + lax.broadcasted_iota(jnp.int32, (RPT, 1), 0))
        real = rowid < N

        @pl.when(i == 0)
        def _():
            spos_ref[...] = jnp.zeros_like(spos_ref)

        @pl.when(i < nh)
        def _():
            nm = jnp.where(real, norm, 0.0)
            spos_ref[...] += jnp.sum(a_ref[...] * nm, axis=0, keepdims=True)

        @pl.when(i >= nh)
        def _():
            summary = jax.nn.sigmoid(
                jnp.dot(spos_ref[...] / N, w1_ref[...],
                        preferred_element_type=jnp.float32) + b1_ref[...])
            ws = lax.dot_general(summary, wd_ref[...],
                                 (((1,), (1,)), ((), ())),
                                 preferred_element_type=jnp.float32)
            h = jnp.dot(a_ref[...] * norm, w1_ref[...],
                        preferred_element_type=jnp.float32) + b1_ref[...]
            logits = lax.dot_general(h, ws, (((1,), (1,)), ((), ())),
                                     preferred_element_type=jnp.float32)
            sign = jnp.where(i < 2 * nh, -1.0, 1.0)
            val = jnp.where(real, jax.nn.softplus(sign * logits), 0.0)
            part = (jnp.sum(val) / N).reshape(1, 1)

            @pl.when(i == nh)
            def _():
                o_ref[...] = jnp.zeros_like(o_ref)

            o_ref[...] += part

    def agg_idx(i):
        return (jnp.where(i < nh, i, i - nh), 0)

    def deg_idx(i):
        return (i % nh, 0)

    return pl.pallas_call(
        body,
        grid=(3 * nh,),
        in_specs=[
            pl.BlockSpec((RPT, D), agg_idx),
            pl.BlockSpec((RPT, 1), deg_idx),
            pl.BlockSpec((D, D), lambda i: (0, 0)),
            pl.BlockSpec((1, D), lambda i: (0, 0)),
            pl.BlockSpec((D, D), lambda i: (0, 0)),
        ],
        out_specs=pl.BlockSpec((1, 1), lambda i: (0, 0)),
        out_shape=jax.ShapeDtypeStruct((1, 1), jnp.float32),
        scratch_shapes=[pltpu.VMEM((1, D), jnp.float32)],
    )(agg2, deg2d, W1, b1.reshape(1, D), Wd)


def kernel(features, edge_index, W0, b0, W1, b1, Wd):
    N, D = features.shape
    E = edge_index.shape[1]
    NP = (N // 256 + 1) * 256
    RPT = NP // NS
    CPT = -(-E // (NS * CH * GRP)) * GRP
    CPD = -(-E // (NW * CH * GDEG)) * GDEG
    PC = -(-(NP // NW) // CH)

    src = edge_index[0].astype(jnp.int32)
    dst = edge_index[1].astype(jnp.int32)
    perm = jax.random.permutation(jax.random.key(42), N).astype(jnp.int32)

    src16 = jnp.pad(src, (0, NS * CPT * CH - E)).reshape(NS, CPT, CH)
    srcw = jnp.concatenate([src16, src16 + NP]).reshape(NW, CPT, CH)
    dst16 = jnp.pad(dst, (0, NS * CPT * CH - E),
                    constant_values=N).reshape(NS, CPT, CH)
    dstd = jnp.pad(dst, (0, NW * CPD * CH - E),
                   constant_values=N).reshape(NW, CPD, CH)
    permp = jnp.pad(perm, (0, NW * PC * CH - N)).reshape(NW, PC, CH)
    zerosD = jnp.zeros((RPT, D), jnp.float32)
    onesD = jnp.ones((CH, D), jnp.float32)

    deg2 = _make_sc_deg_perm(NP, D, CPD, PC, RPT)(dstd, onesD, zerosD)
    fneg_raw = _make_sc_perm(NP, D, PC)(permp, features)
    deg2d = (deg2[:NP, 0] + deg2[NP:, 0]).reshape(NP, 1)

    fpad = jnp.pad(features, ((0, NP - N), (0, 0)))
    f2 = jnp.concatenate([fpad, fneg_raw[:NP]], axis=0)

    RB = 4 * RPT
    prop = _make_sc_prop(NP, D, CPT, RPT)
    x0 = _tc_prescale(f2, deg2d, NP, D, RB)
    agg1 = prop(x0, srcw, dst16, zerosD)
    x1 = _tc_layer(agg1, deg2d, W0, b0, NP, D, RB)
    agg2 = prop(x1, srcw, dst16, zerosD)
    loss = _tc_loss(agg2, deg2d, W1, b1, Wd, N, NP, D, RB)
    return loss[0, 0]


# speedup vs baseline: 1.1394x; 1.0049x over previous
import functools

import jax
import jax.numpy as jnp
from jax import lax
from jax.experimental import pallas as pl
from jax.experimental.pallas import tpu as pltpu
from jax.experimental.pallas import tpu_sc as plsc

NC = 2
NS = 16
NW = NC * NS
CH = 128


def _sc_mesh():
    return plsc.VectorSubcoreMesh(
        core_axis_name="c", subcore_axis_name="s", num_cores=NC, num_subcores=NS
    )


GDEG = 8


def _make_sc_deg_perm(NP, D, CPD, PC, RPT):
    def body(dstd, ones_h, zeros_h, deg_out, deg_sh, didx, ones_v, sem):
        c = lax.axis_index("c")
        s = lax.axis_index("s")
        w = c * NS + s
        row0 = s * RPT
        pltpu.sync_copy(ones_h, ones_v)
        pltpu.sync_copy(zeros_h, deg_sh.at[pl.ds(row0, RPT)])
        plsc.subcore_barrier()

        def gb(g, carry):
            pltpu.sync_copy(dstd.at[w, pl.ds(g * GDEG, GDEG)], didx)

            def jb(j, carry2):
                pltpu.async_copy(ones_v, deg_sh.at[didx.at[j]], sem, add=True)
                return carry2

            lax.fori_loop(0, GDEG, jb, carry)

            def db(j, carry2):
                pltpu.make_async_copy(ones_v, deg_sh.at[pl.ds(0, CH)],
                                      sem).wait()
                return carry2

            return lax.fori_loop(0, GDEG, db, carry)

        lax.fori_loop(0, CPD // GDEG, gb, 0)
        plsc.subcore_barrier()
        pltpu.sync_copy(deg_sh.at[pl.ds(row0, RPT)],
                        deg_out.at[pl.ds(c * NP + row0, RPT)])

    return pl.kernel(
        body,
        out_type=jax.ShapeDtypeStruct((NC * NP, D), jnp.float32),
        mesh=_sc_mesh(),
        scratch_types=[
            pltpu.VMEM_SHARED((NP, D), jnp.float32),
            pltpu.VMEM((GDEG, CH), jnp.int32),
            pltpu.VMEM((CH, D), jnp.float32),
            pltpu.SemaphoreType.DMA,
        ],
    )


def _make_sc_perm(NP, D, PC):
    def body(permp, feat, fneg_out, pidx, fr0, fr1, sem):
        c = lax.axis_index("c")
        s = lax.axis_index("s")
        w = c * NS + s
        pltpu.sync_copy(permp.at[w], pidx.at[pl.ds(0, PC)])
        bufs = [fr0, fr1]
        for j in range(PC):
            fr = bufs[j % 2]
            pltpu.async_copy(feat.at[pidx.at[j]], fr, sem).wait()
            pltpu.sync_copy(fr, fneg_out.at[pl.ds(w * PC * CH + j * CH, CH)])

    return pl.kernel(
        body,
        out_type=jax.ShapeDtypeStruct((NW * PC * CH, D), jnp.float32),
        mesh=_sc_mesh(),
        scratch_types=[
            pltpu.VMEM((8, CH), jnp.int32),
            pltpu.VMEM((CH, D), jnp.float32),
            pltpu.VMEM((CH, D), jnp.float32),
            pltpu.SemaphoreType.DMA,
        ],
    )


GRP = 32


def _make_sc_prop(NP, D, CPT, RPT):
    def body(x2, srcw, dst16, zerosD, agg_out,
             agg_sh, sidx, didx, rows0, rows1, gsem, ssem):
        c = lax.axis_index("c")
        s = lax.axis_index("s")
        w = c * NS + s
        row0 = s * RPT
        pltpu.sync_copy(zerosD, agg_sh.at[pl.ds(row0, RPT)])
        plsc.subcore_barrier()

        def drain_g(buf):
            pltpu.make_async_copy(zerosD.at[pl.ds(0, CH)], buf, gsem).wait()

        def drain_s():
            pltpu.make_async_copy(rows0, agg_sh.at[pl.ds(0, CH)], ssem).wait()

        def gbody(g, carry):
            pltpu.sync_copy(srcw.at[w, pl.ds(g * GRP, GRP)], sidx)
            pltpu.sync_copy(dst16.at[s, pl.ds(g * GRP, GRP)], didx)
            pltpu.async_copy(x2.at[sidx.at[0]], rows0, gsem)

            def pair(t, carry2):
                j = 2 * t
                drain_g(rows0)

                @pl.when(t > 0)
                def _():
                    drain_s()

                pltpu.async_copy(x2.at[sidx.at[j + 1]], rows1, gsem)
                pltpu.async_copy(rows0, agg_sh.at[didx.at[j]], ssem, add=True)
                drain_g(rows1)
                drain_s()

                @pl.when(j + 2 < GRP)
                def _():
                    pltpu.async_copy(x2.at[sidx.at[j + 2]], rows0, gsem)

                pltpu.async_copy(rows1, agg_sh.at[didx.at[j + 1]], ssem,
                                 add=True)
                return carry2

            lax.fori_loop(0, GRP // 2, pair, carry)
            drain_s()
            return carry

        lax.fori_loop(0, CPT // GRP, gbody, 0)
        plsc.subcore_barrier()
        pltpu.sync_copy(agg_sh.at[pl.ds(row0, RPT)],
                        agg_out.at[pl.ds(c * NP + row0, RPT)])

    return pl.kernel(
        body,
        out_type=jax.ShapeDtypeStruct((NC * NP, D), jnp.float32),
        mesh=_sc_mesh(),
        scratch_types=[
            pltpu.VMEM_SHARED((NP, D), jnp.float32),
            pltpu.VMEM((GRP, CH), jnp.int32),
            pltpu.VMEM((GRP, CH), jnp.int32),
            pltpu.VMEM((CH, D), jnp.float32),
            pltpu.VMEM((CH, D), jnp.float32),
            pltpu.SemaphoreType.DMA,
            pltpu.SemaphoreType.DMA,
        ],
    )


def _norm_from(d_ref):
    deg = d_ref[...]
    return jnp.where(deg > 0.0, lax.rsqrt(jnp.maximum(deg, 1.0)), 0.0)


def _tc_prescale(f2, deg2d, NP, D, RPT):
    def body(f_ref, d_ref, o_ref):
        o_ref[...] = f_ref[...] * _norm_from(d_ref)

    nb = (2 * NP) // RPT
    return pl.pallas_call(
        body,
        grid=(nb,),
        in_specs=[
            pl.BlockSpec((RPT, D), lambda i: (i, 0)),
            pl.BlockSpec((RPT, 1), lambda i: (i % (nb // 2), 0)),
        ],
        out_specs=pl.BlockSpec((RPT, D), lambda i: (i, 0)),
        out_shape=jax.ShapeDtypeStruct((2 * NP, D), jnp.float32),
    )(f2, deg2d)


def _tc_layer(agg, deg2d, W, b, NP, D, RPT):
    def body(a_ref, d_ref, w_ref, b_ref, o_ref):
        norm = _norm_from(d_ref)
        h = jnp.dot(a_ref[...] * norm, w_ref[...],
                    preferred_element_type=jnp.float32) + b_ref[...]
        o_ref[...] = jnp.maximum(h, 0.0) * norm

    nb = (2 * NP) // RPT
    return pl.pallas_call(
        body,
        grid=(nb,),
        in_specs=[
            pl.BlockSpec((RPT, D), lambda i: (i, 0)),
            pl.BlockSpec((RPT, 1), lambda i: (i % (nb // 2), 0)),
            pl.BlockSpec((D, D), lambda i: (0, 0)),
            pl.BlockSpec((1, D), lambda i: (0, 0)),
        ],
        out_specs=pl.BlockSpec((RPT, D), lambda i: (i, 0)),
        out_shape=jax.ShapeDtypeStruct((2 * NP, D), jnp.float32),
    )(agg, deg2d, W, b.reshape(1, D))


def _tc_loss(agg2, deg2d, W1, b1, Wd, N, NP, D, RPT):
    nh = NP // RPT

    def body(a_ref, d_ref, w1_ref, b1_ref, wd_ref, o_ref, spos_ref):
        i = pl.program_id(0)
        norm = _norm_from(d_ref)
        rowid = ((i % nh) * RPT
                 + lax.broadcasted_iota(jnp.int32, (RPT, 1), 0))
        real = rowid < N

        @pl.when(i == 0)
        def _():
            spos_ref[...] = jnp.zeros_like(spos_ref)

        @pl.when(i < nh)
        def _():
            nm = jnp.where(real, norm, 0.0)
            spos_ref[...] += jnp.sum(a_ref[...] * nm, axis=0, keepdims=True)

        @pl.when(i >= nh)
        def _():
            summary = jax.nn.sigmoid(
                jnp.dot(spos_ref[...] / N, w1_ref[...],
                        preferred_element_type=jnp.float32) + b1_ref[...])
            ws = lax.dot_general(summary, wd_ref[...],
                                 (((1,), (1,)), ((), ())),
                                 preferred_element_type=jnp.float32)
            h = jnp.dot(a_ref[...] * norm, w1_ref[...],
                        preferred_element_type=jnp.float32) + b1_ref[...]
            logits = lax.dot_general(h, ws, (((1,), (1,)), ((), ())),
                                     preferred_element_type=jnp.float32)
            sign = jnp.where(i < 2 * nh, -1.0, 1.0)
            val = jnp.where(real, jax.nn.softplus(sign * logits), 0.0)
            part = (jnp.sum(val) / N).reshape(1, 1)

            @pl.when(i == nh)
            def _():
                o_ref[...] = jnp.zeros_like(o_ref)

            o_ref[...] += part

    def agg_idx(i):
        return (jnp.where(i < nh, i, i - nh), 0)

    def deg_idx(i):
        return (i % nh, 0)

    return pl.pallas_call(
        body,
        grid=(3 * nh,),
        in_specs=[
            pl.BlockSpec((RPT, D), agg_idx),
            pl.BlockSpec((RPT, 1), deg_idx),
            pl.BlockSpec((D, D), lambda i: (0, 0)),
            pl.BlockSpec((1, D), lambda i: (0, 0)),
            pl.BlockSpec((D, D), lambda i: (0, 0)),
        ],
        out_specs=pl.BlockSpec((1, 1), lambda i: (0, 0)),
        out_shape=jax.ShapeDtypeStruct((1, 1), jnp.float32),
        scratch_shapes=[pltpu.VMEM((1, D), jnp.float32)],
    )(agg2, deg2d, W1, b1.reshape(1, D), Wd)


def kernel(features, edge_index, W0, b0, W1, b1, Wd):
    N, D = features.shape
    E = edge_index.shape[1]
    NP = (N // 256 + 1) * 256
    RPT = NP // NS
    CPT = -(-E // (NS * CH * GRP)) * GRP
    CPD = -(-E // (NW * CH * GDEG)) * GDEG
    PC = -(-(NP // NW) // CH)

    src = edge_index[0].astype(jnp.int32)
    dst = edge_index[1].astype(jnp.int32)
    perm = jax.random.permutation(jax.random.key(42), N).astype(jnp.int32)

    src16 = jnp.pad(src, (0, NS * CPT * CH - E)).reshape(NS, CPT, CH)
    srcw = jnp.concatenate([src16, src16 + NP]).reshape(NW, CPT, CH)
    dst16 = jnp.pad(dst, (0, NS * CPT * CH - E),
                    constant_values=N).reshape(NS, CPT, CH)
    dstd = jnp.pad(dst, (0, NW * CPD * CH - E),
                   constant_values=N).reshape(NW, CPD, CH)
    permp = jnp.pad(perm, (0, NW * PC * CH - N)).reshape(NW, PC, CH)
    zerosD = jnp.zeros((RPT, D), jnp.float32)
    onesD = jnp.ones((CH, D), jnp.float32)

    deg2 = _make_sc_deg_perm(NP, D, CPD, PC, RPT)(dstd, onesD, zerosD)
    fneg_raw = _make_sc_perm(NP, D, PC)(permp, features)
    deg2d = (deg2[:NP, 0] + deg2[NP:, 0]).reshape(NP, 1)

    fpad = jnp.pad(features, ((0, NP - N), (0, 0)))
    f2 = jnp.concatenate([fpad, fneg_raw[:NP]], axis=0)

    RB = 8 * RPT
    prop = _make_sc_prop(NP, D, CPT, RPT)
    x0 = _tc_prescale(f2, deg2d, NP, D, RB)
    agg1 = prop(x0, srcw, dst16, zerosD)
    x1 = _tc_layer(agg1, deg2d, W0, b0, NP, D, RB)
    agg2 = prop(x1, srcw, dst16, zerosD)
    loss = _tc_loss(agg2, deg2d, W1, b1, Wd, N, NP, D, RB)
    return loss[0, 0]


# speedup vs baseline: 1.1438x; 1.0039x over previous
import functools

import jax
import jax.numpy as jnp
from jax import lax
from jax.experimental import pallas as pl
from jax.experimental.pallas import tpu as pltpu
from jax.experimental.pallas import tpu_sc as plsc

NC = 2
NS = 16
NW = NC * NS
CH = 128


def _sc_mesh():
    return plsc.VectorSubcoreMesh(
        core_axis_name="c", subcore_axis_name="s", num_cores=NC, num_subcores=NS
    )


GDEG = 80


def _make_sc_deg_perm(NP, D, CPD, PC, RPT):
    def body(dstd, ones_h, zeros_h, deg_out, deg_sh, didx, ones_v, sem):
        c = lax.axis_index("c")
        s = lax.axis_index("s")
        w = c * NS + s
        row0 = s * RPT
        pltpu.sync_copy(ones_h, ones_v)
        pltpu.sync_copy(zeros_h, deg_sh.at[pl.ds(row0, RPT)])
        plsc.subcore_barrier()

        def gb(g, carry):
            pltpu.sync_copy(dstd.at[w, pl.ds(g * GDEG, GDEG)], didx)

            def jb(j, carry2):
                pltpu.async_copy(ones_v, deg_sh.at[didx.at[j]], sem, add=True)
                return carry2

            lax.fori_loop(0, GDEG, jb, carry)

            def db(j, carry2):
                pltpu.make_async_copy(ones_v, deg_sh.at[pl.ds(0, CH)],
                                      sem).wait()
                return carry2

            return lax.fori_loop(0, GDEG, db, carry)

        lax.fori_loop(0, CPD // GDEG, gb, 0)
        plsc.subcore_barrier()
        pltpu.sync_copy(deg_sh.at[pl.ds(row0, RPT)],
                        deg_out.at[pl.ds(c * NP + row0, RPT)])

    return pl.kernel(
        body,
        out_type=jax.ShapeDtypeStruct((NC * NP, D), jnp.float32),
        mesh=_sc_mesh(),
        scratch_types=[
            pltpu.VMEM_SHARED((NP, D), jnp.float32),
            pltpu.VMEM((GDEG, CH), jnp.int32),
            pltpu.VMEM((CH, D), jnp.float32),
            pltpu.SemaphoreType.DMA,
        ],
    )


def _make_sc_perm(NP, D, PC):
    def body(permp, feat, fneg_out, pidx, fr0, fr1, sem):
        c = lax.axis_index("c")
        s = lax.axis_index("s")
        w = c * NS + s
        pltpu.sync_copy(permp.at[w], pidx.at[pl.ds(0, PC)])
        bufs = [fr0, fr1]
        for j in range(PC):
            fr = bufs[j % 2]
            pltpu.async_copy(feat.at[pidx.at[j]], fr, sem).wait()
            pltpu.sync_copy(fr, fneg_out.at[pl.ds(w * PC * CH + j * CH, CH)])

    return pl.kernel(
        body,
        out_type=jax.ShapeDtypeStruct((NW * PC * CH, D), jnp.float32),
        mesh=_sc_mesh(),
        scratch_types=[
            pltpu.VMEM((8, CH), jnp.int32),
            pltpu.VMEM((CH, D), jnp.float32),
            pltpu.VMEM((CH, D), jnp.float32),
            pltpu.SemaphoreType.DMA,
        ],
    )


GRP = 32


def _make_sc_prop(NP, D, CPT, RPT):
    def body(x2, srcw, dst16, zerosD, agg_out,
             agg_sh, sidx, didx, rows0, rows1, gsem, ssem):
        c = lax.axis_index("c")
        s = lax.axis_index("s")
        w = c * NS + s
        row0 = s * RPT
        pltpu.sync_copy(zerosD, agg_sh.at[pl.ds(row0, RPT)])
        plsc.subcore_barrier()

        def drain_g(buf):
            pltpu.make_async_copy(zerosD.at[pl.ds(0, CH)], buf, gsem).wait()

        def drain_s():
            pltpu.make_async_copy(rows0, agg_sh.at[pl.ds(0, CH)], ssem).wait()

        def gbody(g, carry):
            pltpu.sync_copy(srcw.at[w, pl.ds(g * GRP, GRP)], sidx)
            pltpu.sync_copy(dst16.at[s, pl.ds(g * GRP, GRP)], didx)
            pltpu.async_copy(x2.at[sidx.at[0]], rows0, gsem)

            def pair(t, carry2):
                j = 2 * t
                drain_g(rows0)

                @pl.when(t > 0)
                def _():
                    drain_s()

                pltpu.async_copy(x2.at[sidx.at[j + 1]], rows1, gsem)
                pltpu.async_copy(rows0, agg_sh.at[didx.at[j]], ssem, add=True)
                drain_g(rows1)
                drain_s()

                @pl.when(j + 2 < GRP)
                def _():
                    pltpu.async_copy(x2.at[sidx.at[j + 2]], rows0, gsem)

                pltpu.async_copy(rows1, agg_sh.at[didx.at[j + 1]], ssem,
                                 add=True)
                return carry2

            lax.fori_loop(0, GRP // 2, pair, carry)
            drain_s()
            return carry

        lax.fori_loop(0, CPT // GRP, gbody, 0)
        plsc.subcore_barrier()
        pltpu.sync_copy(agg_sh.at[pl.ds(row0, RPT)],
                        agg_out.at[pl.ds(c * NP + row0, RPT)])

    return pl.kernel(
        body,
        out_type=jax.ShapeDtypeStruct((NC * NP, D), jnp.float32),
        mesh=_sc_mesh(),
        scratch_types=[
            pltpu.VMEM_SHARED((NP, D), jnp.float32),
            pltpu.VMEM((GRP, CH), jnp.int32),
            pltpu.VMEM((GRP, CH), jnp.int32),
            pltpu.VMEM((CH, D), jnp.float32),
            pltpu.VMEM((CH, D), jnp.float32),
            pltpu.SemaphoreType.DMA,
            pltpu.SemaphoreType.DMA,
        ],
    )


def _norm_from(d_ref):
    deg = d_ref[...]
    return jnp.where(deg > 0.0, lax.rsqrt(jnp.maximum(deg, 1.0)), 0.0)


def _tc_prescale(f2, deg2d, NP, D, RPT):
    def body(f_ref, d_ref, o_ref):
        o_ref[...] = f_ref[...] * _norm_from(d_ref)

    nb = (2 * NP) // RPT
    return pl.pallas_call(
        body,
        grid=(nb,),
        in_specs=[
            pl.BlockSpec((RPT, D), lambda i: (i, 0)),
            pl.BlockSpec((RPT, 1), lambda i: (i % (nb // 2), 0)),
        ],
        out_specs=pl.BlockSpec((RPT, D), lambda i: (i, 0)),
        out_shape=jax.ShapeDtypeStruct((2 * NP, D), jnp.float32),
    )(f2, deg2d)


def _tc_layer(agg, deg2d, W, b, NP, D, RPT):
    def body(a_ref, d_ref, w_ref, b_ref, o_ref):
        norm = _norm_from(d_ref)
        h = jnp.dot(a_ref[...] * norm, w_ref[...],
                    preferred_element_type=jnp.float32) + b_ref[...]
        o_ref[...] = jnp.maximum(h, 0.0) * norm

    nb = (2 * NP) // RPT
    return pl.pallas_call(
        body,
        grid=(nb,),
        in_specs=[
            pl.BlockSpec((RPT, D), lambda i: (i, 0)),
            pl.BlockSpec((RPT, 1), lambda i: (i % (nb // 2), 0)),
            pl.BlockSpec((D, D), lambda i: (0, 0)),
            pl.BlockSpec((1, D), lambda i: (0, 0)),
        ],
        out_specs=pl.BlockSpec((RPT, D), lambda i: (i, 0)),
        out_shape=jax.ShapeDtypeStruct((2 * NP, D), jnp.float32),
    )(agg, deg2d, W, b.reshape(1, D))


def _tc_loss(agg2, deg2d, W1, b1, Wd, N, NP, D, RPT):
    nh = NP // RPT

    def body(a_ref, d_ref, w1_ref, b1_ref, wd_ref, o_ref, spos_ref):
        i = pl.program_id(0)
        norm = _norm_from(d_ref)
        rowid = ((i % nh) * RPT
                 + lax.broadcasted_iota(jnp.int32, (RPT, 1), 0))
        real = rowid < N

        @pl.when(i == 0)
        def _():
            spos_ref[...] = jnp.zeros_like(spos_ref)

        @pl.when(i < nh)
        def _():
            nm = jnp.where(real, norm, 0.0)
            spos_ref[...] += jnp.sum(a_ref[...] * nm, axis=0, keepdims=True)

        @pl.when(i >= nh)
        def _():
            summary = jax.nn.sigmoid(
                jnp.dot(spos_ref[...] / N, w1_ref[...],
                        preferred_element_type=jnp.float32) + b1_ref[...])
            ws = lax.dot_general(summary, wd_ref[...],
                                 (((1,), (1,)), ((), ())),
                                 preferred_element_type=jnp.float32)
            h = jnp.dot(a_ref[...] * norm, w1_ref[...],
                        preferred_element_type=jnp.float32) + b1_ref[...]
            logits = lax.dot_general(h, ws, (((1,), (1,)), ((), ())),
                                     preferred_element_type=jnp.float32)
            sign = jnp.where(i < 2 * nh, -1.0, 1.0)
            val = jnp.where(real, jax.nn.softplus(sign * logits), 0.0)
            part = (jnp.sum(val) / N).reshape(1, 1)

            @pl.when(i == nh)
            def _():
                o_ref[...] = jnp.zeros_like(o_ref)

            o_ref[...] += part

    def agg_idx(i):
        return (jnp.where(i < nh, i, i - nh), 0)

    def deg_idx(i):
        return (i % nh, 0)

    return pl.pallas_call(
        body,
        grid=(3 * nh,),
        in_specs=[
            pl.BlockSpec((RPT, D), agg_idx),
            pl.BlockSpec((RPT, 1), deg_idx),
            pl.BlockSpec((D, D), lambda i: (0, 0)),
            pl.BlockSpec((1, D), lambda i: (0, 0)),
            pl.BlockSpec((D, D), lambda i: (0, 0)),
        ],
        out_specs=pl.BlockSpec((1, 1), lambda i: (0, 0)),
        out_shape=jax.ShapeDtypeStruct((1, 1), jnp.float32),
        scratch_shapes=[pltpu.VMEM((1, D), jnp.float32)],
    )(agg2, deg2d, W1, b1.reshape(1, D), Wd)


def kernel(features, edge_index, W0, b0, W1, b1, Wd):
    N, D = features.shape
    E = edge_index.shape[1]
    NP = (N // 256 + 1) * 256
    RPT = NP // NS
    CPT = -(-E // (NS * CH * GRP)) * GRP
    CPD = -(-E // (NW * CH * GDEG)) * GDEG
    PC = -(-(NP // NW) // CH)

    src = edge_index[0].astype(jnp.int32)
    dst = edge_index[1].astype(jnp.int32)
    perm = jax.random.permutation(jax.random.key(42), N).astype(jnp.int32)

    src16 = jnp.pad(src, (0, NS * CPT * CH - E)).reshape(NS, CPT, CH)
    srcw = jnp.concatenate([src16, src16 + NP]).reshape(NW, CPT, CH)
    dst16 = jnp.pad(dst, (0, NS * CPT * CH - E),
                    constant_values=N).reshape(NS, CPT, CH)
    dstd = jnp.pad(dst, (0, NW * CPD * CH - E),
                   constant_values=N).reshape(NW, CPD, CH)
    permp = jnp.pad(perm, (0, NW * PC * CH - N)).reshape(NW, PC, CH)
    zerosD = jnp.zeros((RPT, D), jnp.float32)
    onesD = jnp.ones((CH, D), jnp.float32)

    deg2 = _make_sc_deg_perm(NP, D, CPD, PC, RPT)(dstd, onesD, zerosD)
    fneg_raw = _make_sc_perm(NP, D, PC)(permp, features)
    deg2d = (deg2[:NP, 0] + deg2[NP:, 0]).reshape(NP, 1)

    fpad = jnp.pad(features, ((0, NP - N), (0, 0)))
    f2 = jnp.concatenate([fpad, fneg_raw[:NP]], axis=0)

    RB = 8 * RPT
    prop = _make_sc_prop(NP, D, CPT, RPT)
    x0 = _tc_prescale(f2, deg2d, NP, D, RB)
    agg1 = prop(x0, srcw, dst16, zerosD)
    x1 = _tc_layer(agg1, deg2d, W0, b0, NP, D, RB)
    agg2 = prop(x1, srcw, dst16, zerosD)
    loss = _tc_loss(agg2, deg2d, W1, b1, Wd, N, NP, D, RB)
    return loss[0, 0]


# speedup vs baseline: 1.1459x; 1.0018x over previous
import functools

import jax
import jax.numpy as jnp
from jax import lax
from jax.experimental import pallas as pl
from jax.experimental.pallas import tpu as pltpu
from jax.experimental.pallas import tpu_sc as plsc

NC = 2
NS = 16
NW = NC * NS
CH = 128


def _sc_mesh():
    return plsc.VectorSubcoreMesh(
        core_axis_name="c", subcore_axis_name="s", num_cores=NC, num_subcores=NS
    )


GDEG = 80


def _make_sc_deg_perm(NP, D, CPD, PC, RPT):
    def body(dstd, ones_h, zeros_h, deg_out, deg_sh, didx, ones_v, sem):
        c = lax.axis_index("c")
        s = lax.axis_index("s")
        w = c * NS + s
        row0 = s * RPT
        pltpu.sync_copy(ones_h, ones_v)
        pltpu.sync_copy(zeros_h, deg_sh.at[pl.ds(row0, RPT)])
        plsc.subcore_barrier()

        def gb(g, carry):
            pltpu.sync_copy(dstd.at[w, pl.ds(g * GDEG, GDEG)], didx)

            def jb(j, carry2):
                pltpu.async_copy(ones_v, deg_sh.at[didx.at[j]], sem, add=True)
                return carry2

            lax.fori_loop(0, GDEG, jb, carry)

            def db(j, carry2):
                pltpu.make_async_copy(ones_v, deg_sh.at[pl.ds(0, CH)],
                                      sem).wait()
                return carry2

            return lax.fori_loop(0, GDEG, db, carry)

        lax.fori_loop(0, CPD // GDEG, gb, 0)
        plsc.subcore_barrier()
        pltpu.sync_copy(deg_sh.at[pl.ds(row0, RPT)],
                        deg_out.at[pl.ds(c * NP + row0, RPT)])

    return pl.kernel(
        body,
        out_type=jax.ShapeDtypeStruct((NC * NP, D), jnp.float32),
        mesh=_sc_mesh(),
        scratch_types=[
            pltpu.VMEM_SHARED((NP, D), jnp.float32),
            pltpu.VMEM((GDEG, CH), jnp.int32),
            pltpu.VMEM((CH, D), jnp.float32),
            pltpu.SemaphoreType.DMA,
        ],
    )


def _make_sc_perm(NP, D, PC):
    def body(permp, feat, fneg_out, pidx, fr0, fr1, sem):
        c = lax.axis_index("c")
        s = lax.axis_index("s")
        w = c * NS + s
        pltpu.sync_copy(permp.at[w], pidx.at[pl.ds(0, PC)])
        bufs = [fr0, fr1]
        for j in range(PC):
            fr = bufs[j % 2]
            pltpu.async_copy(feat.at[pidx.at[j]], fr, sem).wait()
            pltpu.sync_copy(fr, fneg_out.at[pl.ds(w * PC * CH + j * CH, CH)])

    return pl.kernel(
        body,
        out_type=jax.ShapeDtypeStruct((NW * PC * CH, D), jnp.float32),
        mesh=_sc_mesh(),
        scratch_types=[
            pltpu.VMEM((8, CH), jnp.int32),
            pltpu.VMEM((CH, D), jnp.float32),
            pltpu.VMEM((CH, D), jnp.float32),
            pltpu.SemaphoreType.DMA,
        ],
    )


GRP = 16


def _make_sc_prop(NP, D, CPT, RPT):
    NG = CPT // GRP

    def body(x2, srcw, dst16, zerosD, agg_out,
             agg_sh, sidx0, didx0, sidx1, didx1, rows0, rows1,
             gsem, ssem, isem):
        c = lax.axis_index("c")
        s = lax.axis_index("s")
        w = c * NS + s
        row0 = s * RPT
        pltpu.sync_copy(zerosD, agg_sh.at[pl.ds(row0, RPT)])
        pltpu.sync_copy(srcw.at[w, pl.ds(0, GRP)], sidx0)
        pltpu.sync_copy(dst16.at[s, pl.ds(0, GRP)], didx0)
        plsc.subcore_barrier()

        def drain_g(buf):
            pltpu.make_async_copy(zerosD.at[pl.ds(0, CH)], buf, gsem).wait()

        def drain_s():
            pltpu.make_async_copy(rows0, agg_sh.at[pl.ds(0, CH)], ssem).wait()

        def drain_i(buf):
            pltpu.make_async_copy(srcw.at[w, pl.ds(0, GRP)], buf, isem).wait()

        def prefetch(g, sbuf, dbuf):
            pltpu.async_copy(srcw.at[w, pl.ds(g * GRP, GRP)], sbuf, isem)
            pltpu.async_copy(dst16.at[s, pl.ds(g * GRP, GRP)], dbuf, isem)

        def run_group(sidx, didx):
            pltpu.async_copy(x2.at[sidx.at[0]], rows0, gsem)

            def pair(t, carry2):
                j = 2 * t
                drain_g(rows0)

                @pl.when(t > 0)
                def _():
                    drain_s()

                pltpu.async_copy(x2.at[sidx.at[j + 1]], rows1, gsem)
                pltpu.async_copy(rows0, agg_sh.at[didx.at[j]], ssem, add=True)
                drain_g(rows1)
                drain_s()

                @pl.when(j + 2 < GRP)
                def _():
                    pltpu.async_copy(x2.at[sidx.at[j + 2]], rows0, gsem)

                pltpu.async_copy(rows1, agg_sh.at[didx.at[j + 1]], ssem,
                                 add=True)
                return carry2

            lax.fori_loop(0, GRP // 2, pair, 0)
            drain_s()

        def g2(t, carry):
            g = 2 * t
            prefetch(g + 1, sidx1, didx1)
            run_group(sidx0, didx0)
            drain_i(sidx1)
            drain_i(didx1)

            @pl.when(g + 2 < NG)
            def _():
                prefetch(g + 2, sidx0, didx0)

            run_group(sidx1, didx1)

            @pl.when(g + 2 < NG)
            def _():
                drain_i(sidx0)
                drain_i(didx0)

            return carry

        lax.fori_loop(0, NG // 2, g2, 0)
        plsc.subcore_barrier()
        pltpu.sync_copy(agg_sh.at[pl.ds(row0, RPT)],
                        agg_out.at[pl.ds(c * NP + row0, RPT)])

    return pl.kernel(
        body,
        out_type=jax.ShapeDtypeStruct((NC * NP, D), jnp.float32),
        mesh=_sc_mesh(),
        scratch_types=[
            pltpu.VMEM_SHARED((NP, D), jnp.float32),
            pltpu.VMEM((GRP, CH), jnp.int32),
            pltpu.VMEM((GRP, CH), jnp.int32),
            pltpu.VMEM((GRP, CH), jnp.int32),
            pltpu.VMEM((GRP, CH), jnp.int32),
            pltpu.VMEM((CH, D), jnp.float32),
            pltpu.VMEM((CH, D), jnp.float32),
            pltpu.SemaphoreType.DMA,
            pltpu.SemaphoreType.DMA,
            pltpu.SemaphoreType.DMA,
        ],
    )


def _norm_from(d_ref):
    deg = d_ref[...]
    return jnp.where(deg > 0.0, lax.rsqrt(jnp.maximum(deg, 1.0)), 0.0)


def _tc_prescale(f2, deg2d, NP, D, RPT):
    def body(f_ref, d_ref, o_ref):
        o_ref[...] = f_ref[...] * _norm_from(d_ref)

    nb = (2 * NP) // RPT
    return pl.pallas_call(
        body,
        grid=(nb,),
        in_specs=[
            pl.BlockSpec((RPT, D), lambda i: (i, 0)),
            pl.BlockSpec((RPT, 1), lambda i: (i % (nb // 2), 0)),
        ],
        out_specs=pl.BlockSpec((RPT, D), lambda i: (i, 0)),
        out_shape=jax.ShapeDtypeStruct((2 * NP, D), jnp.float32),
    )(f2, deg2d)


def _tc_layer(agg, deg2d, W, b, NP, D, RPT):
    def body(a_ref, d_ref, w_ref, b_ref, o_ref):
        norm = _norm_from(d_ref)
        h = jnp.dot(a_ref[...] * norm, w_ref[...],
                    preferred_element_type=jnp.float32) + b_ref[...]
        o_ref[...] = jnp.maximum(h, 0.0) * norm

    nb = (2 * NP) // RPT
    return pl.pallas_call(
        body,
        grid=(nb,),
        in_specs=[
            pl.BlockSpec((RPT, D), lambda i: (i, 0)),
            pl.BlockSpec((RPT, 1), lambda i: (i % (nb // 2), 0)),
            pl.BlockSpec((D, D), lambda i: (0, 0)),
            pl.BlockSpec((1, D), lambda i: (0, 0)),
        ],
        out_specs=pl.BlockSpec((RPT, D), lambda i: (i, 0)),
        out_shape=jax.ShapeDtypeStruct((2 * NP, D), jnp.float32),
    )(agg, deg2d, W, b.reshape(1, D))


def _tc_loss(agg2, deg2d, W1, b1, Wd, N, NP, D, RPT):
    nh = NP // RPT

    def body(a_ref, d_ref, w1_ref, b1_ref, wd_ref, o_ref, spos_ref):
        i = pl.program_id(0)
        norm = _norm_from(d_ref)
        rowid = ((i % nh) * RPT
                 + lax.broadcasted_iota(jnp.int32, (RPT, 1), 0))
        real = rowid < N

        @pl.when(i == 0)
        def _():
            spos_ref[...] = jnp.zeros_like(spos_ref)

        @pl.when(i < nh)
        def _():
            nm = jnp.where(real, norm, 0.0)
            spos_ref[...] += jnp.sum(a_ref[...] * nm, axis=0, keepdims=True)

        @pl.when(i >= nh)
        def _():
            summary = jax.nn.sigmoid(
                jnp.dot(spos_ref[...] / N, w1_ref[...],
                        preferred_element_type=jnp.float32) + b1_ref[...])
            ws = lax.dot_general(summary, wd_ref[...],
                                 (((1,), (1,)), ((), ())),
                                 preferred_element_type=jnp.float32)
            h = jnp.dot(a_ref[...] * norm, w1_ref[...],
                        preferred_element_type=jnp.float32) + b1_ref[...]
            logits = lax.dot_general(h, ws, (((1,), (1,)), ((), ())),
                                     preferred_element_type=jnp.float32)
            sign = jnp.where(i < 2 * nh, -1.0, 1.0)
            val = jnp.where(real, jax.nn.softplus(sign * logits), 0.0)
            part = (jnp.sum(val) / N).reshape(1, 1)

            @pl.when(i == nh)
            def _():
                o_ref[...] = jnp.zeros_like(o_ref)

            o_ref[...] += part

    def agg_idx(i):
        return (jnp.where(i < nh, i, i - nh), 0)

    def deg_idx(i):
        return (i % nh, 0)

    return pl.pallas_call(
        body,
        grid=(3 * nh,),
        in_specs=[
            pl.BlockSpec((RPT, D), agg_idx),
            pl.BlockSpec((RPT, 1), deg_idx),
            pl.BlockSpec((D, D), lambda i: (0, 0)),
            pl.BlockSpec((1, D), lambda i: (0, 0)),
            pl.BlockSpec((D, D), lambda i: (0, 0)),
        ],
        out_specs=pl.BlockSpec((1, 1), lambda i: (0, 0)),
        out_shape=jax.ShapeDtypeStruct((1, 1), jnp.float32),
        scratch_shapes=[pltpu.VMEM((1, D), jnp.float32)],
    )(agg2, deg2d, W1, b1.reshape(1, D), Wd)


def kernel(features, edge_index, W0, b0, W1, b1, Wd):
    N, D = features.shape
    E = edge_index.shape[1]
    NP = (N // 256 + 1) * 256
    RPT = NP // NS
    CPT = -(-E // (NS * CH * GRP)) * GRP
    CPD = -(-E // (NW * CH * GDEG)) * GDEG
    PC = -(-(NP // NW) // CH)

    src = edge_index[0].astype(jnp.int32)
    dst = edge_index[1].astype(jnp.int32)
    perm = jax.random.permutation(jax.random.key(42), N).astype(jnp.int32)

    src16 = jnp.pad(src, (0, NS * CPT * CH - E)).reshape(NS, CPT, CH)
    srcw = jnp.concatenate([src16, src16 + NP]).reshape(NW, CPT, CH)
    dst16 = jnp.pad(dst, (0, NS * CPT * CH - E),
                    constant_values=N).reshape(NS, CPT, CH)
    dstd = jnp.pad(dst, (0, NW * CPD * CH - E),
                   constant_values=N).reshape(NW, CPD, CH)
    permp = jnp.pad(perm, (0, NW * PC * CH - N)).reshape(NW, PC, CH)
    zerosD = jnp.zeros((RPT, D), jnp.float32)
    onesD = jnp.ones((CH, D), jnp.float32)

    deg2 = _make_sc_deg_perm(NP, D, CPD, PC, RPT)(dstd, onesD, zerosD)
    fneg_raw = _make_sc_perm(NP, D, PC)(permp, features)
    deg2d = (deg2[:NP, 0] + deg2[NP:, 0]).reshape(NP, 1)

    fpad = jnp.pad(features, ((0, NP - N), (0, 0)))
    f2 = jnp.concatenate([fpad, fneg_raw[:NP]], axis=0)

    RB = 8 * RPT
    prop = _make_sc_prop(NP, D, CPT, RPT)
    x0 = _tc_prescale(f2, deg2d, NP, D, RB)
    agg1 = prop(x0, srcw, dst16, zerosD)
    x1 = _tc_layer(agg1, deg2d, W0, b0, NP, D, RB)
    agg2 = prop(x1, srcw, dst16, zerosD)
    loss = _tc_loss(agg2, deg2d, W1, b1, Wd, N, NP, D, RB)
    return loss[0, 0]
